# baseline (speedup 1.0000x reference)
import functools

import jax
import jax.numpy as jnp
from jax import lax
from jax.experimental import pallas as pl
from jax.experimental.pallas import tpu as pltpu
from jax.experimental.pallas import tpu_sc as plsc

NC = 2
NS = 16
NW = NC * NS
K = 112
SB = 6
NSLOT = 3
NB = 3


def _sc_mesh():
    return plsc.VectorSubcoreMesh(core_axis_name="c", subcore_axis_name="s")


def _make_deg_kernel(NP, EPW):
    rows_pt = NP // NS

    @functools.partial(
        pl.kernel,
        out_type=jax.ShapeDtypeStruct((NC, NP), jnp.float32),
        mesh=_sc_mesh(),
        compiler_params=pltpu.CompilerParams(needs_layout_passes=False),
        scratch_types=[
            pltpu.VMEM((EPW,), jnp.int32),
            pltpu.VMEM((NP,), jnp.float32),
            pltpu.VMEM((NS, rows_pt), jnp.float32),
            pltpu.VMEM((rows_pt,), jnp.float32),
            pltpu.VMEM_SHARED((NS, NP), jnp.float32),
        ],
    )
    def deg_kernel(dst_hbm, zeros_hbm, out_hbm, dst_l, cnt, tmp, res, shr):
        cid = lax.axis_index("c")
        sid = lax.axis_index("s")
        wid = sid * NC + cid
        pltpu.sync_copy(dst_hbm.at[wid], dst_l)
        pltpu.sync_copy(zeros_hbm, cnt)
        ones = jnp.ones((16,), jnp.float32)

        @pl.loop(0, EPW // 16)
        def _(i):
            iv = dst_l[pl.ds(i * 16, 16)]
            plsc.addupdate_scatter(cnt, [iv], ones)

        pltpu.sync_copy(cnt, shr.at[sid])
        plsc.subcore_barrier()
        pltpu.sync_copy(shr.at[:, pl.ds(sid * rows_pt, rows_pt)], tmp)

        @pl.loop(0, rows_pt // 16)
        def _(c):
            tot = jnp.zeros((16,), jnp.float32)
            for t in range(NS):
                tot = tot + tmp[t, pl.ds(c * 16, 16)]
            res[pl.ds(c * 16, 16)] = tot

        pltpu.sync_copy(res, out_hbm.at[cid, pl.ds(sid * rows_pt, rows_pt)])

    return deg_kernel


def _make_agg_kernel(N, D, NP, NCH):
    rows_pt = NP // NS
    nseg = NCH // SB

    @functools.partial(
        pl.kernel,
        out_type=jax.ShapeDtypeStruct((NC, NP, D), jnp.float32),
        mesh=_sc_mesh(),
        scratch_types=[
            pltpu.VMEM((NSLOT, SB, K), jnp.int32),
            pltpu.VMEM((NSLOT, SB, K), jnp.int32),
            [pltpu.VMEM((K, D), jnp.float32) for _ in range(NB)],
            pltpu.VMEM_SHARED((NP, D), jnp.float32),
            [pltpu.SemaphoreType.DMA for _ in range(NB)],
            [pltpu.SemaphoreType.DMA for _ in range(NB)],
            pltpu.SemaphoreType.DMA,
            pltpu.SemaphoreType.DMA,
        ],
    )
    def agg_kernel(hp_hbm, src_hbm, dst_hbm, zeros_hbm, out_hbm,
                   src_l, dst_l, bufs, acc, gsem, ssem, semis, semid):
        cid = lax.axis_index("c")
        sid = lax.axis_index("s")
        wid = sid * NC + cid

        def load_seg_async(g, slot):
            pltpu.async_copy(src_hbm.at[wid, g], src_l.at[slot], semis)
            pltpu.async_copy(dst_hbm.at[wid, g], dst_l.at[slot], semid)

        def wait_seg(g, slot):
            pltpu.make_async_copy(src_hbm.at[wid, g],
                                  src_l.at[slot], semis).wait()
            pltpu.make_async_copy(dst_hbm.at[wid, g],
                                  dst_l.at[slot], semid).wait()

        def src_idx(j):
            g = j // SB
            return src_l.at[g % NSLOT, j - g * SB]

        def dst_idx(j):
            g = j // SB
            return dst_l.at[g % NSLOT, j - g * SB]

        def start_gather(j, b):
            pltpu.async_copy(hp_hbm.at[src_idx(j)], bufs[b], gsem[b])

        pltpu.sync_copy(src_hbm.at[wid, 0], src_l.at[0])
        pltpu.sync_copy(dst_hbm.at[wid, 0], dst_l.at[0])
        if nseg > 1:
            load_seg_async(1, 1 % NSLOT)
        start_gather(0, 0)
        start_gather(1, 1)
        pltpu.sync_copy(zeros_hbm, acc.at[pl.ds(sid * rows_pt, rows_pt)])
        plsc.subcore_barrier()

        @pl.loop(0, NCH, step=NB)
        def _(jj):
            @pl.when(jnp.logical_and(jj % SB == SB // 2,
                                     jj <= NCH - SB - NB))
            def _():
                m = jj // SB
                wait_seg(m + 1, (m + 1) % NSLOT)

                @pl.when(jj <= NCH - 2 * SB - NB)
                def _():
                    load_seg_async(m + 2, (m + 2) % NSLOT)

            for b in range(NB):
                j = jj + b
                pltpu.make_async_copy(hp_hbm.at[src_idx(j)], bufs[b],
                                      gsem[b]).wait()
                pltpu.async_copy(bufs[b], acc.at[dst_idx(j)], ssem[b],
                                 add=True)
                b2 = (b + 2) % NB

                @pl.when(j + 2 < NCH)
                def _():
                    @pl.when(j >= 1)
                    def _():
                        pltpu.make_async_copy(
                            bufs[b2], acc.at[dst_idx(j - 1)], ssem[b2]
                        ).wait()

                    start_gather(j + 2, b2)

        for j in range(NCH - NB, NCH):
            pltpu.make_async_copy(bufs[j % NB], acc.at[dst_idx(j)],
                                  ssem[j % NB]).wait()
        plsc.subcore_barrier()
        pltpu.sync_copy(acc.at[pl.ds(sid * rows_pt, rows_pt)],
                        out_hbm.at[cid, pl.ds(sid * rows_pt, rows_pt)])

    return agg_kernel


def _dense0_body(x_ref, w0_ref, wr_ref, br_ref, da_ref, db_ref,
                 hp_ref, r_ref, dis_ref):
    deg = da_ref[...] + db_ref[...] + 1.0
    dis = lax.rsqrt(deg)
    x = x_ref[...]
    hp_ref[...] = jnp.dot(x, w0_ref[...], preferred_element_type=jnp.float32) * dis
    r_ref[...] = jnp.dot(x, wr_ref[...], preferred_element_type=jnp.float32) + br_ref[...]
    dis_ref[...] = dis


def _dense1_body(a0_ref, a1_ref, hp_ref, r_ref, dis_ref, b_ref,
                 w1_ref, wr_ref, br_ref, hp1_ref, r1_ref):
    dis = dis_ref[...]
    agg = a0_ref[0] + a1_ref[0] + hp_ref[...]
    x1 = jnp.maximum(dis * agg + b_ref[...] + r_ref[...], 0.0)
    hp1_ref[...] = jnp.dot(x1, w1_ref[...], preferred_element_type=jnp.float32) * dis
    r1_ref[...] = jnp.dot(x1, wr_ref[...], preferred_element_type=jnp.float32) + br_ref[...]


def _dense2_body(a0_ref, a1_ref, hp_ref, r_ref, dis_ref, b_ref, y_ref):
    dis = dis_ref[...]
    agg = a0_ref[0] + a1_ref[0] + hp_ref[...]
    y_ref[...] = jnp.maximum(dis * agg + b_ref[...] + r_ref[...], 0.0)


def kernel(x, edge_index, W0, b0, W1, b1, Wr, br):
    N, D = x.shape
    E = edge_index.shape[1]

    NP = ((N + 1 + 255) // 256) * 256
    rows_pt = NP // NS

    per_w = NW * K
    NCH = -(-(-(-E // per_w)) // SB) * SB
    E_pad = NCH * per_w
    src = edge_index[0]
    dst = edge_index[1]
    if E_pad != E:
        pad = E_pad - E
        src = jnp.concatenate([src, jnp.zeros((pad,), src.dtype)])
        dst = jnp.concatenate([dst, jnp.full((pad,), NP - 1, dst.dtype)])
    src3 = src.reshape(NW, NCH // SB, SB, K)
    dst3 = dst.reshape(NW, NCH // SB, SB, K)

    zeros_d = jnp.zeros((rows_pt, D), jnp.float32)

    EPW = -(-(-(-E // NW)) // 16) * 16
    dstd = edge_index[1]
    if EPW * NW != E:
        dstd = jnp.concatenate(
            [dstd, jnp.full((EPW * NW - E,), NP - 1, dstd.dtype)])
    dst2 = dstd.reshape(NW, EPW)

    deg_out = _make_deg_kernel(NP, EPW)(dst2, jnp.zeros((NP,), jnp.float32))
    degA = deg_out[0, :N].reshape(N, 1)
    degB = deg_out[1, :N].reshape(N, 1)

    BN = 400 if N % 400 == 0 else (8 if N % 8 == 0 else N)
    grid = (N // BN,)
    row_spec = pl.BlockSpec((BN, D), lambda i: (i, 0))
    col_spec = pl.BlockSpec((BN, 1), lambda i: (i, 0))
    full_spec = pl.BlockSpec((D, D), lambda i: (0, 0))
    bias_spec = pl.BlockSpec((1, D), lambda i: (0, 0))
    accA_spec = pl.BlockSpec((1, BN, D), lambda i: (0, i, 0))
    accB_spec = pl.BlockSpec((1, BN, D), lambda i: (1, i, 0))

    br2 = br.reshape(1, D)
    b02 = b0.reshape(1, D)
    b12 = b1.reshape(1, D)

    hp0, r0, dis = pl.pallas_call(
        _dense0_body,
        grid=grid,
        in_specs=[row_spec, full_spec, full_spec, bias_spec, col_spec, col_spec],
        out_specs=[row_spec, row_spec, col_spec],
        out_shape=[
            jax.ShapeDtypeStruct((N, D), jnp.float32),
            jax.ShapeDtypeStruct((N, D), jnp.float32),
            jax.ShapeDtypeStruct((N, 1), jnp.float32),
        ],
    )(x, W0, Wr, br2, degA, degB)

    agg_call = _make_agg_kernel(N, D, NP, NCH)

    acc0 = agg_call(hp0, src3, dst3, zeros_d)

    hp1, r1 = pl.pallas_call(
        _dense1_body,
        grid=grid,
        in_specs=[accA_spec, accB_spec, row_spec, row_spec, col_spec,
                  bias_spec, full_spec, full_spec, bias_spec],
        out_specs=[row_spec, row_spec],
        out_shape=[
            jax.ShapeDtypeStruct((N, D), jnp.float32),
            jax.ShapeDtypeStruct((N, D), jnp.float32),
        ],
    )(acc0, acc0, hp0, r0, dis, b02, W1, Wr, br2)

    acc1 = agg_call(hp1, src3, dst3, zeros_d)

    y = pl.pallas_call(
        _dense2_body,
        grid=grid,
        in_specs=[accA_spec, accB_spec, row_spec, row_spec, col_spec, bias_spec],
        out_specs=row_spec,
        out_shape=jax.ShapeDtypeStruct((N, D), jnp.float32),
    )(acc1, acc1, hp1, r1, dis, b12)

    return y


# speedup vs baseline: 1.8945x; 1.8945x over previous
import functools

import jax
import jax.numpy as jnp
from jax import lax
from jax.experimental import pallas as pl
from jax.experimental.pallas import tpu as pltpu
from jax.experimental.pallas import tpu_sc as plsc

NC = 2
NS = 16
NW = NC * NS
K = 125
SB = 8
NSLOT = 4


def _sc_mesh():
    return plsc.VectorSubcoreMesh(core_axis_name="c", subcore_axis_name="s")


def _make_deg_kernel(NP, EPW):
    rows_pt = NP // NS

    @functools.partial(
        pl.kernel,
        out_type=jax.ShapeDtypeStruct((NC, NP), jnp.float32),
        mesh=_sc_mesh(),
        compiler_params=pltpu.CompilerParams(needs_layout_passes=False),
        scratch_types=[
            pltpu.VMEM((EPW,), jnp.int32),
            pltpu.VMEM((NP,), jnp.float32),
            pltpu.VMEM((NS, rows_pt), jnp.float32),
            pltpu.VMEM((rows_pt,), jnp.float32),
            pltpu.VMEM_SHARED((NS, NP), jnp.float32),
        ],
    )
    def deg_kernel(dst_hbm, zeros_hbm, out_hbm, dst_l, cnt, tmp, res, shr):
        cid = lax.axis_index("c")
        sid = lax.axis_index("s")
        wid = sid * NC + cid
        pltpu.sync_copy(dst_hbm.at[wid], dst_l)
        pltpu.sync_copy(zeros_hbm, cnt)
        ones = jnp.ones((16,), jnp.float32)

        @pl.loop(0, EPW // 16)
        def _(i):
            iv = dst_l[pl.ds(i * 16, 16)]
            plsc.addupdate_scatter(cnt, [iv], ones)

        pltpu.sync_copy(cnt, shr.at[sid])
        plsc.subcore_barrier()
        pltpu.sync_copy(shr.at[:, pl.ds(sid * rows_pt, rows_pt)], tmp)

        @pl.loop(0, rows_pt // 16)
        def _(c):
            tot = jnp.zeros((16,), jnp.float32)
            for t in range(NS):
                tot = tot + tmp[t, pl.ds(c * 16, 16)]
            res[pl.ds(c * 16, 16)] = tot

        pltpu.sync_copy(res, out_hbm.at[cid, pl.ds(sid * rows_pt, rows_pt)])

    return deg_kernel


def _make_agg_kernel(N, D, NP, NCH):
    rows_pt = NP // NS
    nseg = NCH // SB

    @functools.partial(
        pl.kernel,
        out_type=jax.ShapeDtypeStruct((NC, NP, D), jnp.float32),
        mesh=_sc_mesh(),
        scratch_types=[
            pltpu.VMEM((NSLOT, SB, K), jnp.int32),
            pltpu.VMEM((NSLOT, SB, K), jnp.int32),
            pltpu.VMEM((K, D), jnp.float32),
            pltpu.VMEM((K, D), jnp.float32),
            pltpu.VMEM_SHARED((NP, D), jnp.float32),
            pltpu.SemaphoreType.DMA,
            pltpu.SemaphoreType.DMA,
            pltpu.SemaphoreType.DMA,
            pltpu.SemaphoreType.DMA,
        ],
    )
    def agg_kernel(hp_hbm, src_hbm, dst_hbm, zeros_hbm, out_hbm,
                   src_l, dst_l, buf0, buf1, acc, sem0, sem1, semis, semid):
        cid = lax.axis_index("c")
        sid = lax.axis_index("s")
        wid = sid * NC + cid

        def load_seg_async(g, slot):
            pltpu.async_copy(src_hbm.at[wid, pl.ds(g * SB, SB)],
                             src_l.at[slot], semis)
            pltpu.async_copy(dst_hbm.at[wid, pl.ds(g * SB, SB)],
                             dst_l.at[slot], semid)

        def wait_seg(g, slot):
            pltpu.make_async_copy(src_hbm.at[wid, pl.ds(g * SB, SB)],
                                  src_l.at[slot], semis).wait()
            pltpu.make_async_copy(dst_hbm.at[wid, pl.ds(g * SB, SB)],
                                  dst_l.at[slot], semid).wait()

        def start_gather(j, buf, sem):
            g = j // SB
            pltpu.async_copy(hp_hbm.at[src_l.at[g % NSLOT, j - g * SB]],
                             buf, sem)

        pltpu.sync_copy(src_hbm.at[wid, pl.ds(0, SB)], src_l.at[0])
        pltpu.sync_copy(dst_hbm.at[wid, pl.ds(0, SB)], dst_l.at[0])
        if nseg > 1:
            load_seg_async(1, 1 % NSLOT)
        if nseg > 2:
            load_seg_async(2, 2 % NSLOT)
        start_gather(0, buf0, sem0)
        start_gather(1, buf1, sem1)
        pltpu.sync_copy(zeros_hbm, acc.at[pl.ds(sid * rows_pt, rows_pt)])
        plsc.subcore_barrier()

        @pl.loop(0, NCH, step=2)
        def _(jj):
            @pl.when(jnp.logical_and(jj % SB == SB - 2, jj < NCH - SB))
            def _():
                g_next = jj // SB + 1
                wait_seg(g_next, g_next % NSLOT)

                @pl.when(jj < NCH - 3 * SB)
                def _():
                    load_seg_async(g_next + 2, (g_next + 2) % NSLOT)

            for b, (buf, sem) in enumerate(((buf0, sem0), (buf1, sem1))):
                j = jj + b
                g = j // SB
                pltpu.make_async_copy(
                    hp_hbm.at[src_l.at[g % NSLOT, j - g * SB]], buf, sem
                ).wait()
                pltpu.sync_copy(buf, acc.at[dst_l.at[g % NSLOT, j - g * SB]],
                                add=True)

                @pl.when(j + 2 < NCH)
                def _():
                    start_gather(j + 2, buf, sem)

        plsc.subcore_barrier()
        pltpu.sync_copy(acc.at[pl.ds(sid * rows_pt, rows_pt)],
                        out_hbm.at[cid, pl.ds(sid * rows_pt, rows_pt)])

    return agg_kernel


def _dense0_body(x_ref, w0_ref, wr_ref, br_ref, da_ref, db_ref,
                 hp_ref, r_ref, dis_ref):
    deg = da_ref[...] + db_ref[...] + 1.0
    dis = lax.rsqrt(deg)
    x = x_ref[...]
    hp_ref[...] = jnp.dot(x, w0_ref[...], preferred_element_type=jnp.float32) * dis
    r_ref[...] = jnp.dot(x, wr_ref[...], preferred_element_type=jnp.float32) + br_ref[...]
    dis_ref[...] = dis


def _dense1_body(a0_ref, a1_ref, hp_ref, r_ref, dis_ref, b_ref,
                 w1_ref, wr_ref, br_ref, hp1_ref, r1_ref):
    dis = dis_ref[...]
    agg = a0_ref[0] + a1_ref[0] + hp_ref[...]
    x1 = jnp.maximum(dis * agg + b_ref[...] + r_ref[...], 0.0)
    hp1_ref[...] = jnp.dot(x1, w1_ref[...], preferred_element_type=jnp.float32) * dis
    r1_ref[...] = jnp.dot(x1, wr_ref[...], preferred_element_type=jnp.float32) + br_ref[...]


def _dense2_body(a0_ref, a1_ref, hp_ref, r_ref, dis_ref, b_ref, y_ref):
    dis = dis_ref[...]
    agg = a0_ref[0] + a1_ref[0] + hp_ref[...]
    y_ref[...] = jnp.maximum(dis * agg + b_ref[...] + r_ref[...], 0.0)


def kernel(x, edge_index, W0, b0, W1, b1, Wr, br):
    N, D = x.shape
    E = edge_index.shape[1]

    NP = ((N + 1 + 255) // 256) * 256
    rows_pt = NP // NS

    per_w = NW * K
    NCH = -(-(-(-E // per_w)) // SB) * SB
    E_pad = NCH * per_w
    src = edge_index[0]
    dst = edge_index[1]
    if E_pad != E:
        pad = E_pad - E
        src = jnp.concatenate([src, jnp.zeros((pad,), src.dtype)])
        dst = jnp.concatenate([dst, jnp.full((pad,), NP - 1, dst.dtype)])
    src3 = src.reshape(NW, NCH, K)
    dst3 = dst.reshape(NW, NCH, K)

    zeros_d = jnp.zeros((rows_pt, D), jnp.float32)

    EPW = -(-(-(-E // NW)) // 16) * 16
    dstd = edge_index[1]
    if EPW * NW != E:
        dstd = jnp.concatenate(
            [dstd, jnp.full((EPW * NW - E,), NP - 1, dstd.dtype)])
    dst2 = dstd.reshape(NW, EPW)

    deg_out = _make_deg_kernel(NP, EPW)(dst2, jnp.zeros((NP,), jnp.float32))
    degA = deg_out[0, :N].reshape(N, 1)
    degB = deg_out[1, :N].reshape(N, 1)

    BN = 2000 if N % 2000 == 0 else (400 if N % 400 == 0 else (8 if N % 8 == 0 else N))
    grid = (N // BN,)
    row_spec = pl.BlockSpec((BN, D), lambda i: (i, 0))
    col_spec = pl.BlockSpec((BN, 1), lambda i: (i, 0))
    full_spec = pl.BlockSpec((D, D), lambda i: (0, 0))
    bias_spec = pl.BlockSpec((1, D), lambda i: (0, 0))
    accA_spec = pl.BlockSpec((1, BN, D), lambda i: (0, i, 0))
    accB_spec = pl.BlockSpec((1, BN, D), lambda i: (1, i, 0))

    br2 = br.reshape(1, D)
    b02 = b0.reshape(1, D)
    b12 = b1.reshape(1, D)

    hp0, r0, dis = pl.pallas_call(
        _dense0_body,
        grid=grid,
        in_specs=[row_spec, full_spec, full_spec, bias_spec, col_spec, col_spec],
        out_specs=[row_spec, row_spec, col_spec],
        out_shape=[
            jax.ShapeDtypeStruct((N, D), jnp.float32),
            jax.ShapeDtypeStruct((N, D), jnp.float32),
            jax.ShapeDtypeStruct((N, 1), jnp.float32),
        ],
    )(x, W0, Wr, br2, degA, degB)

    agg_call = _make_agg_kernel(N, D, NP, NCH)

    acc0 = agg_call(hp0, src3, dst3, zeros_d)

    hp1, r1 = pl.pallas_call(
        _dense1_body,
        grid=grid,
        in_specs=[accA_spec, accB_spec, row_spec, row_spec, col_spec,
                  bias_spec, full_spec, full_spec, bias_spec],
        out_specs=[row_spec, row_spec],
        out_shape=[
            jax.ShapeDtypeStruct((N, D), jnp.float32),
            jax.ShapeDtypeStruct((N, D), jnp.float32),
        ],
    )(acc0, acc0, hp0, r0, dis, b02, W1, Wr, br2)

    acc1 = agg_call(hp1, src3, dst3, zeros_d)

    y = pl.pallas_call(
        _dense2_body,
        grid=grid,
        in_specs=[accA_spec, accB_spec, row_spec, row_spec, col_spec, bias_spec],
        out_specs=row_spec,
        out_shape=jax.ShapeDtypeStruct((N, D), jnp.float32),
    )(acc1, acc1, hp1, r1, dis, b12)

    return y


# speedup vs baseline: 1.8947x; 1.0001x over previous
import functools

import jax
import jax.numpy as jnp
from jax import lax
from jax.experimental import pallas as pl
from jax.experimental.pallas import tpu as pltpu
from jax.experimental.pallas import tpu_sc as plsc

NC = 2
NS = 16
NW = NC * NS
K = 125
SB = 8
NSLOT = 4


def _sc_mesh():
    return plsc.VectorSubcoreMesh(core_axis_name="c", subcore_axis_name="s")


def _make_deg_kernel(NP, EPW):
    rows_pt = NP // NS

    @functools.partial(
        pl.kernel,
        out_type=jax.ShapeDtypeStruct((NC, NP), jnp.float32),
        mesh=_sc_mesh(),
        compiler_params=pltpu.CompilerParams(needs_layout_passes=False),
        scratch_types=[
            pltpu.VMEM((EPW,), jnp.int32),
            pltpu.VMEM((NP,), jnp.float32),
            pltpu.VMEM((NS, rows_pt), jnp.float32),
            pltpu.VMEM((rows_pt,), jnp.float32),
            pltpu.VMEM_SHARED((NS, NP), jnp.float32),
        ],
    )
    def deg_kernel(dst_hbm, zeros_hbm, out_hbm, dst_l, cnt, tmp, res, shr):
        cid = lax.axis_index("c")
        sid = lax.axis_index("s")
        wid = sid * NC + cid
        pltpu.sync_copy(dst_hbm.at[wid], dst_l)
        pltpu.sync_copy(zeros_hbm, cnt)
        ones = jnp.ones((16,), jnp.float32)

        @pl.loop(0, EPW // 16)
        def _(i):
            iv = dst_l[pl.ds(i * 16, 16)]
            plsc.addupdate_scatter(cnt, [iv], ones)

        pltpu.sync_copy(cnt, shr.at[sid])
        plsc.subcore_barrier()
        pltpu.sync_copy(shr.at[:, pl.ds(sid * rows_pt, rows_pt)], tmp)

        @pl.loop(0, rows_pt // 16)
        def _(c):
            tot = jnp.zeros((16,), jnp.float32)
            for t in range(NS):
                tot = tot + tmp[t, pl.ds(c * 16, 16)]
            res[pl.ds(c * 16, 16)] = tot

        pltpu.sync_copy(res, out_hbm.at[cid, pl.ds(sid * rows_pt, rows_pt)])

    return deg_kernel


def _make_agg_kernel(N, D, NP, NCH):
    rows_pt = NP // NS
    nseg = NCH // SB

    @functools.partial(
        pl.kernel,
        out_type=jax.ShapeDtypeStruct((NC, NP, D), jnp.float32),
        mesh=_sc_mesh(),
        scratch_types=[
            pltpu.VMEM((NSLOT, SB, K), jnp.int32),
            pltpu.VMEM((NSLOT, SB, K), jnp.int32),
            pltpu.VMEM((K, D), jnp.float32),
            pltpu.VMEM((K, D), jnp.float32),
            pltpu.VMEM_SHARED((NP, D), jnp.float32),
            pltpu.SemaphoreType.DMA,
            pltpu.SemaphoreType.DMA,
            pltpu.SemaphoreType.DMA,
            pltpu.SemaphoreType.DMA,
        ],
    )
    def agg_kernel(hp_hbm, src_hbm, dst_hbm, zeros_hbm, out_hbm,
                   src_l, dst_l, buf0, buf1, acc, sem0, sem1, semis, semid):
        cid = lax.axis_index("c")
        sid = lax.axis_index("s")
        wid = sid * NC + cid

        def load_seg_async(g, slot):
            pltpu.async_copy(src_hbm.at[wid, pl.ds(g * SB, SB)],
                             src_l.at[slot], semis)
            pltpu.async_copy(dst_hbm.at[wid, pl.ds(g * SB, SB)],
                             dst_l.at[slot], semid)

        def wait_seg(g, slot):
            pltpu.make_async_copy(src_hbm.at[wid, pl.ds(g * SB, SB)],
                                  src_l.at[slot], semis).wait()
            pltpu.make_async_copy(dst_hbm.at[wid, pl.ds(g * SB, SB)],
                                  dst_l.at[slot], semid).wait()

        def start_gather(j, buf, sem):
            g = j // SB
            pltpu.async_copy(hp_hbm.at[src_l.at[g % NSLOT, j - g * SB]],
                             buf, sem)

        pltpu.sync_copy(src_hbm.at[wid, pl.ds(0, SB)], src_l.at[0])
        pltpu.sync_copy(dst_hbm.at[wid, pl.ds(0, SB)], dst_l.at[0])
        if nseg > 1:
            load_seg_async(1, 1 % NSLOT)
        if nseg > 2:
            load_seg_async(2, 2 % NSLOT)
        start_gather(0, buf0, sem0)
        start_gather(1, buf1, sem1)
        pltpu.sync_copy(zeros_hbm, acc.at[pl.ds(sid * rows_pt, rows_pt)])
        plsc.subcore_barrier()

        @pl.loop(0, NCH, step=2)
        def _(jj):
            @pl.when(jnp.logical_and(jj % SB == SB - 2, jj < NCH - SB))
            def _():
                g_next = jj // SB + 1
                wait_seg(g_next, g_next % NSLOT)

                @pl.when(jj < NCH - 3 * SB)
                def _():
                    load_seg_async(g_next + 2, (g_next + 2) % NSLOT)

            for b, (buf, sem) in enumerate(((buf0, sem0), (buf1, sem1))):
                j = jj + b
                g = j // SB
                pltpu.make_async_copy(
                    hp_hbm.at[src_l.at[g % NSLOT, j - g * SB]], buf, sem
                ).wait()
                pltpu.sync_copy(buf, acc.at[dst_l.at[g % NSLOT, j - g * SB]],
                                add=True)

                @pl.when(j + 2 < NCH)
                def _():
                    start_gather(j + 2, buf, sem)

        plsc.subcore_barrier()
        pltpu.sync_copy(acc.at[pl.ds(sid * rows_pt, rows_pt)],
                        out_hbm.at[cid, pl.ds(sid * rows_pt, rows_pt)])

    return agg_kernel


def _dense0_body(x_ref, w0_ref, wr_ref, br_ref, da_ref, db_ref,
                 hp_ref, r_ref, dis_ref):
    deg = da_ref[...] + db_ref[...] + 1.0
    dis = lax.rsqrt(deg)
    x = x_ref[...]
    hp_ref[...] = jnp.dot(x, w0_ref[...], preferred_element_type=jnp.float32) * dis
    r_ref[...] = jnp.dot(x, wr_ref[...], preferred_element_type=jnp.float32) + br_ref[...]
    dis_ref[...] = dis


def _dense1_body(a0_ref, a1_ref, hp_ref, r_ref, dis_ref, b_ref,
                 w1_ref, wr_ref, br_ref, hp1_ref, r1_ref):
    dis = dis_ref[...]
    agg = a0_ref[0] + a1_ref[0] + hp_ref[...]
    x1 = jnp.maximum(dis * agg + b_ref[...] + r_ref[...], 0.0)
    hp1_ref[...] = jnp.dot(x1, w1_ref[...], preferred_element_type=jnp.float32) * dis
    r1_ref[...] = jnp.dot(x1, wr_ref[...], preferred_element_type=jnp.float32) + br_ref[...]


def _dense2_body(a0_ref, a1_ref, hp_ref, r_ref, dis_ref, b_ref, y_ref):
    dis = dis_ref[...]
    agg = a0_ref[0] + a1_ref[0] + hp_ref[...]
    y_ref[...] = jnp.maximum(dis * agg + b_ref[...] + r_ref[...], 0.0)


def kernel(x, edge_index, W0, b0, W1, b1, Wr, br):
    N, D = x.shape
    E = edge_index.shape[1]

    NP = ((N + 1 + 255) // 256) * 256
    rows_pt = NP // NS

    per_w = NW * K
    NCH = -(-(-(-E // per_w)) // SB) * SB
    E_pad = NCH * per_w
    src = edge_index[0]
    dst = edge_index[1]
    if E_pad != E:
        pad = E_pad - E
        src = jnp.concatenate([src, jnp.zeros((pad,), src.dtype)])
        dst = jnp.concatenate([dst, jnp.full((pad,), NP - 1, dst.dtype)])
    src3 = src.reshape(NW, NCH, K)
    dst3 = dst.reshape(NW, NCH, K)

    zeros_d = jnp.zeros((rows_pt, D), jnp.float32)

    EPW = -(-(-(-E // NW)) // 16) * 16
    dstd = edge_index[1]
    if EPW * NW != E:
        dstd = jnp.concatenate(
            [dstd, jnp.full((EPW * NW - E,), NP - 1, dstd.dtype)])
    dst2 = dstd.reshape(NW, EPW)

    deg_out = _make_deg_kernel(NP, EPW)(dst2, jnp.zeros((NP,), jnp.float32))
    degA = deg_out[0, :N].reshape(N, 1)
    degB = deg_out[1, :N].reshape(N, 1)

    BN = 5000 if N % 5000 == 0 else (400 if N % 400 == 0 else (8 if N % 8 == 0 else N))
    grid = (N // BN,)
    row_spec = pl.BlockSpec((BN, D), lambda i: (i, 0))
    col_spec = pl.BlockSpec((BN, 1), lambda i: (i, 0))
    full_spec = pl.BlockSpec((D, D), lambda i: (0, 0))
    bias_spec = pl.BlockSpec((1, D), lambda i: (0, 0))
    accA_spec = pl.BlockSpec((1, BN, D), lambda i: (0, i, 0))
    accB_spec = pl.BlockSpec((1, BN, D), lambda i: (1, i, 0))

    br2 = br.reshape(1, D)
    b02 = b0.reshape(1, D)
    b12 = b1.reshape(1, D)

    hp0, r0, dis = pl.pallas_call(
        _dense0_body,
        grid=grid,
        in_specs=[row_spec, full_spec, full_spec, bias_spec, col_spec, col_spec],
        out_specs=[row_spec, row_spec, col_spec],
        out_shape=[
            jax.ShapeDtypeStruct((N, D), jnp.float32),
            jax.ShapeDtypeStruct((N, D), jnp.float32),
            jax.ShapeDtypeStruct((N, 1), jnp.float32),
        ],
    )(x, W0, Wr, br2, degA, degB)

    agg_call = _make_agg_kernel(N, D, NP, NCH)

    acc0 = agg_call(hp0, src3, dst3, zeros_d)

    hp1, r1 = pl.pallas_call(
        _dense1_body,
        grid=grid,
        in_specs=[accA_spec, accB_spec, row_spec, row_spec, col_spec,
                  bias_spec, full_spec, full_spec, bias_spec],
        out_specs=[row_spec, row_spec],
        out_shape=[
            jax.ShapeDtypeStruct((N, D), jnp.float32),
            jax.ShapeDtypeStruct((N, D), jnp.float32),
        ],
    )(acc0, acc0, hp0, r0, dis, b02, W1, Wr, br2)

    acc1 = agg_call(hp1, src3, dst3, zeros_d)

    y = pl.pallas_call(
        _dense2_body,
        grid=grid,
        in_specs=[accA_spec, accB_spec, row_spec, row_spec, col_spec, bias_spec],
        out_specs=row_spec,
        out_shape=jax.ShapeDtypeStruct((N, D), jnp.float32),
    )(acc1, acc1, hp1, r1, dis, b12)

    return y


# speedup vs baseline: 1.8970x; 1.0012x over previous
import functools

import jax
import jax.numpy as jnp
from jax import lax
from jax.experimental import pallas as pl
from jax.experimental.pallas import tpu as pltpu
from jax.experimental.pallas import tpu_sc as plsc

NC = 2
NS = 16
NW = NC * NS
K = 125
SB = 8
NSLOT = 4


def _sc_mesh():
    return plsc.VectorSubcoreMesh(core_axis_name="c", subcore_axis_name="s")


def _make_deg_kernel(NP, EPW):
    rows_pt = NP // NS

    @functools.partial(
        pl.kernel,
        out_type=jax.ShapeDtypeStruct((NC, NP), jnp.float32),
        mesh=_sc_mesh(),
        compiler_params=pltpu.CompilerParams(needs_layout_passes=False),
        scratch_types=[
            pltpu.VMEM((EPW,), jnp.int32),
            pltpu.VMEM((NP,), jnp.float32),
            pltpu.VMEM((NS, rows_pt), jnp.float32),
            pltpu.VMEM((rows_pt,), jnp.float32),
            pltpu.VMEM_SHARED((NS, NP), jnp.float32),
        ],
    )
    def deg_kernel(dst_hbm, zeros_hbm, out_hbm, dst_l, cnt, tmp, res, shr):
        cid = lax.axis_index("c")
        sid = lax.axis_index("s")
        wid = sid * NC + cid
        pltpu.sync_copy(dst_hbm.at[wid], dst_l)
        pltpu.sync_copy(zeros_hbm, cnt)
        ones = jnp.ones((16,), jnp.float32)

        @pl.loop(0, EPW // 16)
        def _(i):
            iv = dst_l[pl.ds(i * 16, 16)]
            plsc.addupdate_scatter(cnt, [iv], ones)

        pltpu.sync_copy(cnt, shr.at[sid])
        plsc.subcore_barrier()
        pltpu.sync_copy(shr.at[:, pl.ds(sid * rows_pt, rows_pt)], tmp)

        @pl.loop(0, rows_pt // 16)
        def _(c):
            tot = jnp.zeros((16,), jnp.float32)
            for t in range(NS):
                tot = tot + tmp[t, pl.ds(c * 16, 16)]
            res[pl.ds(c * 16, 16)] = tot

        pltpu.sync_copy(res, out_hbm.at[cid, pl.ds(sid * rows_pt, rows_pt)])

    return deg_kernel


def _make_agg_kernel(N, D, NP, NCH):
    rows_pt = NP // NS
    nseg = NCH // SB

    @functools.partial(
        pl.kernel,
        out_type=jax.ShapeDtypeStruct((NC, NP, D), jnp.float32),
        mesh=_sc_mesh(),
        scratch_types=[
            pltpu.VMEM((NSLOT, SB, K), jnp.int32),
            pltpu.VMEM((NSLOT, SB, K), jnp.int32),
            pltpu.VMEM((K, D), jnp.float32),
            pltpu.VMEM((K, D), jnp.float32),
            pltpu.VMEM_SHARED((NP, D), jnp.float32),
            pltpu.SemaphoreType.DMA,
            pltpu.SemaphoreType.DMA,
            pltpu.SemaphoreType.DMA,
            pltpu.SemaphoreType.DMA,
        ],
    )
    def agg_kernel(hp_hbm, src_hbm, dst_hbm, zeros_hbm, out_hbm,
                   src_l, dst_l, buf0, buf1, acc, sem0, sem1, semis, semid):
        cid = lax.axis_index("c")
        sid = lax.axis_index("s")
        wid = sid * NC + cid

        def load_seg_async(g, slot):
            pltpu.async_copy(src_hbm.at[wid, pl.ds(g * SB, SB)],
                             src_l.at[slot], semis)
            pltpu.async_copy(dst_hbm.at[wid, pl.ds(g * SB, SB)],
                             dst_l.at[slot], semid)

        def wait_seg(g, slot):
            pltpu.make_async_copy(src_hbm.at[wid, pl.ds(g * SB, SB)],
                                  src_l.at[slot], semis).wait()
            pltpu.make_async_copy(dst_hbm.at[wid, pl.ds(g * SB, SB)],
                                  dst_l.at[slot], semid).wait()

        def start_gather(j, buf, sem):
            g = j // SB
            pltpu.async_copy(hp_hbm.at[src_l.at[g % NSLOT, j - g * SB]],
                             buf, sem)

        pltpu.sync_copy(src_hbm.at[wid, pl.ds(0, SB)], src_l.at[0])
        pltpu.sync_copy(dst_hbm.at[wid, pl.ds(0, SB)], dst_l.at[0])
        if nseg > 1:
            load_seg_async(1, 1 % NSLOT)
        if nseg > 2:
            load_seg_async(2, 2 % NSLOT)
        start_gather(0, buf0, sem0)
        start_gather(1, buf1, sem1)
        pltpu.sync_copy(zeros_hbm, acc.at[pl.ds(sid * rows_pt, rows_pt)])
        plsc.subcore_barrier()

        @pl.loop(0, NCH, step=2)
        def _(jj):
            @pl.when(jnp.logical_and(jj % SB == SB - 2, jj < NCH - SB))
            def _():
                g_next = jj // SB + 1
                wait_seg(g_next, g_next % NSLOT)

                @pl.when(jj < NCH - 3 * SB)
                def _():
                    load_seg_async(g_next + 2, (g_next + 2) % NSLOT)

            for b, (buf, sem) in enumerate(((buf0, sem0), (buf1, sem1))):
                j = jj + b
                g = j // SB
                pltpu.make_async_copy(
                    hp_hbm.at[src_l.at[g % NSLOT, j - g * SB]], buf, sem
                ).wait()
                pltpu.sync_copy(buf, acc.at[dst_l.at[g % NSLOT, j - g * SB]],
                                add=True)

                @pl.when(j + 2 < NCH)
                def _():
                    start_gather(j + 2, buf, sem)

        plsc.subcore_barrier()
        pltpu.sync_copy(acc.at[pl.ds(sid * rows_pt, rows_pt)],
                        out_hbm.at[cid, pl.ds(sid * rows_pt, rows_pt)])

    return agg_kernel


def _dense0_body(x_ref, w0_ref, wr_ref, br_ref, da_ref, db_ref,
                 hp_ref, r_ref, dis_ref):
    deg = da_ref[...] + db_ref[...] + 1.0
    dis = lax.rsqrt(deg)
    x = x_ref[...]
    hp_ref[...] = jnp.dot(x, w0_ref[...], preferred_element_type=jnp.float32) * dis
    r_ref[...] = jnp.dot(x, wr_ref[...], preferred_element_type=jnp.float32) + br_ref[...]
    dis_ref[...] = dis


def _dense1_body(a0_ref, a1_ref, hp_ref, r_ref, dis_ref, b_ref,
                 w1_ref, wr_ref, br_ref, hp1_ref, r1_ref):
    dis = dis_ref[...]
    agg = a0_ref[0] + a1_ref[0] + hp_ref[...]
    x1 = jnp.maximum(dis * agg + b_ref[...] + r_ref[...], 0.0)
    hp1_ref[...] = jnp.dot(x1, w1_ref[...], preferred_element_type=jnp.float32) * dis
    r1_ref[...] = jnp.dot(x1, wr_ref[...], preferred_element_type=jnp.float32) + br_ref[...]


def _dense2_body(a0_ref, a1_ref, hp_ref, r_ref, dis_ref, b_ref, y_ref):
    dis = dis_ref[...]
    agg = a0_ref[0] + a1_ref[0] + hp_ref[...]
    y_ref[...] = jnp.maximum(dis * agg + b_ref[...] + r_ref[...], 0.0)


def kernel(x, edge_index, W0, b0, W1, b1, Wr, br):
    N, D = x.shape
    E = edge_index.shape[1]

    NP = ((N + 1 + 255) // 256) * 256
    rows_pt = NP // NS

    per_w = NW * K
    NCH = -(-(-(-E // per_w)) // SB) * SB
    E_pad = NCH * per_w
    src = edge_index[0]
    dst = edge_index[1]
    if E_pad != E:
        pad = E_pad - E
        src = jnp.concatenate([src, jnp.zeros((pad,), src.dtype)])
        dst = jnp.concatenate([dst, jnp.full((pad,), NP - 1, dst.dtype)])
    src3 = src.reshape(NW, NCH, K)
    dst3 = dst.reshape(NW, NCH, K)

    zeros_d = jnp.zeros((rows_pt, D), jnp.float32)

    EPW = -(-(-(-E // NW)) // 16) * 16
    dstd = edge_index[1]
    if EPW * NW != E:
        dstd = jnp.concatenate(
            [dstd, jnp.full((EPW * NW - E,), NP - 1, dstd.dtype)])
    dst2 = dstd.reshape(NW, EPW)

    deg_out = _make_deg_kernel(NP, EPW)(dst2, jnp.zeros((NP,), jnp.float32))
    degA = deg_out[0, :N].reshape(N, 1)
    degB = deg_out[1, :N].reshape(N, 1)

    BN = 2000 if N % 2000 == 0 else (400 if N % 400 == 0 else (8 if N % 8 == 0 else N))
    grid = (N // BN,)
    row_spec = pl.BlockSpec((BN, D), lambda i: (i, 0))
    col_spec = pl.BlockSpec((BN, 1), lambda i: (i, 0))
    full_spec = pl.BlockSpec((D, D), lambda i: (0, 0))
    bias_spec = pl.BlockSpec((1, D), lambda i: (0, 0))
    accA_spec = pl.BlockSpec((1, BN, D), lambda i: (0, i, 0))
    accB_spec = pl.BlockSpec((1, BN, D), lambda i: (1, i, 0))

    br2 = br.reshape(1, D)
    b02 = b0.reshape(1, D)
    b12 = b1.reshape(1, D)

    hp0, r0, dis = pl.pallas_call(
        _dense0_body,
        grid=grid,
        in_specs=[row_spec, full_spec, full_spec, bias_spec, col_spec, col_spec],
        out_specs=[row_spec, row_spec, col_spec],
        out_shape=[
            jax.ShapeDtypeStruct((N, D), jnp.float32),
            jax.ShapeDtypeStruct((N, D), jnp.float32),
            jax.ShapeDtypeStruct((N, 1), jnp.float32),
        ],
    )(x, W0, Wr, br2, degA, degB)

    agg_call = _make_agg_kernel(N, D, NP, NCH)

    acc0 = agg_call(hp0, src3, dst3, zeros_d)

    hp1, r1 = pl.pallas_call(
        _dense1_body,
        grid=grid,
        in_specs=[accA_spec, accB_spec, row_spec, row_spec, col_spec,
                  bias_spec, full_spec, full_spec, bias_spec],
        out_specs=[row_spec, row_spec],
        out_shape=[
            jax.ShapeDtypeStruct((N, D), jnp.float32),
            jax.ShapeDtypeStruct((N, D), jnp.float32),
        ],
    )(acc0, acc0, hp0, r0, dis, b02, W1, Wr, br2)

    acc1 = agg_call(hp1, src3, dst3, zeros_d)

    y = pl.pallas_call(
        _dense2_body,
        grid=grid,
        in_specs=[accA_spec, accB_spec, row_spec, row_spec, col_spec, bias_spec],
        out_specs=row_spec,
        out_shape=jax.ShapeDtypeStruct((N, D), jnp.float32),
    )(acc1, acc1, hp1, r1, dis, b12)

    return y
